# Initial kernel scaffold; baseline (speedup 1.0000x reference)
import jax
import jax.numpy as jnp
from jax.experimental import pallas as pl


def kernel(h, e, edge_index):
    raise NotImplementedError("write your pallas kernel here")


# speedup vs baseline: 45.2739x; 45.2739x over previous
import functools

import jax
import jax.numpy as jnp
from jax import lax
from jax.experimental import pallas as pl
from jax.experimental.pallas import tpu as pltpu
from jax.experimental.pallas import tpu_sc as plsc

N = 10000
NP = 10240
E = 160000
E2 = 2 * E
DP = 128
NT = 16
EPT = E2 // NT
EC = 160
NCH = EPT // EC
RPT = NP // NT
RC = 64
NRC = RPT // RC

_mesh = plsc.VectorSubcoreMesh(core_axis_name="c", subcore_axis_name="s")

_GATHER_DNUMS = lax.GatherDimensionNumbers(
    offset_dims=(), collapsed_slice_dims=(0,), start_index_map=(0,))


def _bcast_lane(v, j):
    idx = jnp.full((16, 1), j, jnp.int32)
    return lax.gather(v, idx, _GATHER_DNUMS, slice_sizes=(1,),
                      mode=lax.GatherScatterMode.PROMISE_IN_BOUNDS)


def _weights_body(src_hbm, dst_hbm, ew_hbm, w_hbm,
                  esum0, esum1, srcb, dstb, e0b, e1b, s0b, s1b):
    c = lax.axis_index("c")
    s = lax.axis_index("s")
    base = s * EPT
    h0 = (c * 2) * E2
    h1 = (c * 2 + 1) * E2

    @pl.when(s == 0)
    def _():
        zv = jnp.zeros((16,), jnp.float32)

        def zg(i, carry):
            e0b[pl.ds(i * 16, 16)] = zv
            return carry
        lax.fori_loop(0, EC // 16, zg, 0)

        def zcp(i, carry):
            pltpu.sync_copy(e0b, esum0.at[pl.ds(i * EC, EC)])
            pltpu.sync_copy(e0b, esum1.at[pl.ds(i * EC, EC)])
            return carry
        lax.fori_loop(0, NP // EC, zcp, 0)
    plsc.subcore_barrier()

    def chunk1(i, carry):
        off = base + i * EC
        pltpu.sync_copy(dst_hbm.at[pl.ds(off, EC)], dstb)
        pltpu.sync_copy(ew_hbm.at[pl.ds(h0 + off, EC)], e0b)
        pltpu.sync_copy(ew_hbm.at[pl.ds(h1 + off, EC)], e1b)
        pltpu.sync_copy(e0b, esum0.at[dstb], add=True)
        pltpu.sync_copy(e1b, esum1.at[dstb], add=True)
        return carry
    lax.fori_loop(0, NCH, chunk1, 0)
    plsc.subcore_barrier()

    def chunk2(i, carry):
        off = base + i * EC
        pltpu.sync_copy(src_hbm.at[pl.ds(off, EC)], srcb)
        pltpu.sync_copy(dst_hbm.at[pl.ds(off, EC)], dstb)
        pltpu.sync_copy(ew_hbm.at[pl.ds(h0 + off, EC)], e0b)
        pltpu.sync_copy(ew_hbm.at[pl.ds(h1 + off, EC)], e1b)
        pltpu.sync_copy(esum0.at[dstb], s0b)
        pltpu.sync_copy(esum1.at[dstb], s1b)

        def grp(g, carry2):
            sl = pl.ds(g * 16, 16)
            m = srcb[sl] == dstb[sl]
            zero = jnp.zeros((16,), jnp.float32)
            e0b[sl] = jnp.where(m, zero, e0b[sl] / s0b[sl])
            e1b[sl] = jnp.where(m, zero, e1b[sl] / s1b[sl])
            return carry2
        lax.fori_loop(0, EC // 16, grp, 0)
        pltpu.sync_copy(e0b, w_hbm.at[pl.ds(h0 + off, EC)])
        pltpu.sync_copy(e1b, w_hbm.at[pl.ds(h1 + off, EC)])
        return carry
    lax.fori_loop(0, NCH, chunk2, 0)


_weights_call = pl.kernel(
    _weights_body,
    out_type=jax.ShapeDtypeStruct((4 * E2,), jnp.float32),
    mesh=_mesh,
    scratch_types=[
        pltpu.VMEM_SHARED((NP,), jnp.float32),
        pltpu.VMEM_SHARED((NP,), jnp.float32),
        pltpu.VMEM((EC,), jnp.int32),
        pltpu.VMEM((EC,), jnp.int32),
        pltpu.VMEM((EC,), jnp.float32),
        pltpu.VMEM((EC,), jnp.float32),
        pltpu.VMEM((EC,), jnp.float32),
        pltpu.VMEM((EC,), jnp.float32),
    ],
)


def _stage_body(cs, dcoef, z_hbm, x_hbm, acc_hbm, w_hbm, src_hbm, dst_hbm,
                zn_hbm, accn_hbm,
                ysp, rows, srcb, dstb, w0b, w1b, kb, xb, ab):
    c = lax.axis_index("c")
    s = lax.axis_index("s")
    h0 = (c * 2) * E2
    h1 = (c * 2 + 1) * E2

    zv = jnp.zeros((16,), jnp.float32)

    def zg(i, carry):
        kb[i // 8, pl.ds((i % 8) * 16, 16)] = zv
        return carry
    lax.fori_loop(0, RC * 8, zg, 0)

    def zcp(j, carry):
        pltpu.sync_copy(kb, ysp.at[pl.ds(s * RPT + j * RC, RC)])
        return carry
    lax.fori_loop(0, NRC, zcp, 0)
    plsc.subcore_barrier()

    cbase = jnp.full((16,), 0, jnp.int32) + c * NP
    ebase = s * EPT

    def echunk(i, carry):
        off = ebase + i * EC
        pltpu.sync_copy(src_hbm.at[pl.ds(off, EC)], srcb)
        pltpu.sync_copy(dst_hbm.at[pl.ds(off, EC)], dstb)
        pltpu.sync_copy(w_hbm.at[pl.ds(h0 + off, EC)], w0b)
        pltpu.sync_copy(w_hbm.at[pl.ds(h1 + off, EC)], w1b)

        def adj(g, carry2):
            sl = pl.ds(g * 16, 16)
            srcb[sl] = srcb[sl] + cbase
            return carry2
        lax.fori_loop(0, EC // 16, adj, 0)
        pltpu.sync_copy(z_hbm.at[srcb], rows)

        def egrp(g, carry2):
            w0v = w0b[pl.ds(g * 16, 16)]
            w1v = w1b[pl.ds(g * 16, 16)]
            for j in range(16):
                wv0 = _bcast_lane(w0v, j)
                wv1 = _bcast_lane(w1v, j)
                ei = g * 16 + j
                for r in range(8):
                    sl = pl.ds(r * 16, 16)
                    wv = wv0 if r < 4 else wv1
                    rows[ei, sl] = rows[ei, sl] * wv
            return carry2
        lax.fori_loop(0, EC // 16, egrp, 0)
        pltpu.sync_copy(rows, ysp.at[dstb], add=True)
        return carry
    lax.fori_loop(0, NCH, echunk, 0)
    plsc.subcore_barrier()

    csv = jnp.full((16,), cs, jnp.float32)
    dsv = jnp.full((16,), dcoef, jnp.float32)

    def rchunk(j, carry):
        r0 = s * RPT + j * RC
        g0 = c * NP + r0
        pltpu.sync_copy(ysp.at[pl.ds(r0, RC)], kb)
        pltpu.sync_copy(x_hbm.at[pl.ds(g0, RC)], xb)
        pltpu.sync_copy(acc_hbm.at[pl.ds(g0, RC)], ab)

        def g(i, carry2):
            rr = i // 8
            sl = pl.ds((i % 8) * 16, 16)
            kv = kb[rr, sl]
            xb[rr, sl] = xb[rr, sl] + csv * kv
            ab[rr, sl] = ab[rr, sl] + dsv * kv
            return carry2
        lax.fori_loop(0, RC * 8, g, 0)
        pltpu.sync_copy(xb, zn_hbm.at[pl.ds(g0, RC)])
        pltpu.sync_copy(ab, accn_hbm.at[pl.ds(g0, RC)])
        return carry
    lax.fori_loop(0, NRC, rchunk, 0)


def _make_stage(cs, dcoef):
    return pl.kernel(
        functools.partial(_stage_body, cs, dcoef),
        out_type=(
            jax.ShapeDtypeStruct((2 * NP, DP), jnp.float32),
            jax.ShapeDtypeStruct((2 * NP, DP), jnp.float32),
        ),
        mesh=_mesh,
        scratch_types=[
            pltpu.VMEM_SHARED((NP, DP), jnp.float32),
            pltpu.VMEM((EC, DP), jnp.float32),
            pltpu.VMEM((EC,), jnp.int32),
            pltpu.VMEM((EC,), jnp.int32),
            pltpu.VMEM((EC,), jnp.float32),
            pltpu.VMEM((EC,), jnp.float32),
            pltpu.VMEM((RC, DP), jnp.float32),
            pltpu.VMEM((RC, DP), jnp.float32),
            pltpu.VMEM((RC, DP), jnp.float32),
        ],
    )


_stages = [_make_stage(cs, dcoef)
           for cs, dcoef in [(0.5, 1.0 / 6.0), (0.5, 1.0 / 3.0),
                             (1.0, 1.0 / 3.0), (1.0, 1.0 / 6.0)]]


@jax.jit
def kernel(h, e, edge_index):
    src, dst = edge_index[0], edge_index[1]
    src2 = jnp.concatenate([src, dst])
    dst2 = jnp.concatenate([dst, src])
    eh = e[:, :, 0]
    e2 = jnp.concatenate([eh, eh], axis=0)
    ew = e2.T.reshape(4 * E2)
    xh = h.reshape(N, 2, DP).transpose(1, 0, 2)
    pad = jnp.zeros((2, NP - N, DP), jnp.float32)
    x2 = jnp.concatenate([xh, pad], axis=1).reshape(2 * NP, DP)

    w = _weights_call(src2, dst2, ew)
    z, acc = x2, x2
    for st in _stages:
        z, acc = st(z, x2, acc, w, src2, dst2)
    y = acc.reshape(2, NP, DP)[:, :N]
    return y.transpose(1, 0, 2).reshape(N, 256)
